# baseline (speedup 1.0000x reference)
import functools
import math

import jax
import jax.numpy as jnp
from jax import lax
from jax.experimental import pallas as pl
from jax.experimental.pallas import tpu as pltpu
from jax.experimental.pallas import tpu_sc as plsc

VOCAB = 1_000_000
D = 64
SCALE = math.sqrt(D)

_info = plsc.get_sparse_core_info()
NC = _info.num_cores
NS = _info.num_subcores
NW = NC * NS
L = _info.num_lanes

CHUNK = 128
NBUF = 4
NOBUF = 2


def _build(S, T):
    B = S * T
    nrows = B // CHUNK
    spt = S // CHUNK
    per_w = nrows // NW
    assert per_w % NBUF == 0
    nouter = per_w // NBUF

    mesh = plsc.VectorSubcoreMesh(core_axis_name="c", subcore_axis_name="s")

    @functools.partial(
        pl.kernel,
        mesh=mesh,
        compiler_params=pltpu.CompilerParams(
            use_tc_tiling_on_sc=True, needs_layout_passes=False),
        out_type=jax.ShapeDtypeStruct((T * D, S), jnp.float32),
        scratch_types=[
            pltpu.VMEM((per_w, CHUNK), jnp.int32),
            [pltpu.VMEM((CHUNK,), jnp.int32) for _ in range(NBUF)],
            [pltpu.VMEM((CHUNK, 2 * D), jnp.float32) for _ in range(NBUF)],
            [pltpu.VMEM((D, CHUNK), jnp.float32) for _ in range(NOBUF)],
            [pltpu.SemaphoreType.DMA for _ in range(NBUF)],
            [pltpu.SemaphoreType.DMA for _ in range(NOBUF)],
        ],
    )
    def emb(tok_hbm, table_hbm, out_hbm, tok_v, idx2, bin_, bout, gsem, osem):
        wid = lax.axis_index("s") * NC + lax.axis_index("c")
        pltpu.sync_copy(tok_hbm.at[pl.ds(wid * per_w, per_w)], tok_v)
        g0 = wid * per_w

        def gather(j, b):
            for sv in range(CHUNK // L):
                sl = pl.ds(sv * L, L)
                idx2[b][sl] = lax.shift_right_logical(tok_v[j, sl], 1)
            pltpu.async_copy(table_hbm.at[idx2[b]], bin_[b], gsem[b])

        def put(j, b):
            g = g0 + j
            t = g // spt
            s0 = (g % spt) * CHUNK
            pltpu.async_copy(
                bout[b], out_hbm.at[pl.ds(t * D, D), pl.ds(s0, CHUNK)], osem[b])

        def put_wait(b):
            pltpu.make_async_copy(
                bout[b], out_hbm.at[pl.ds(0, D), pl.ds(0, CHUNK)],
                osem[b]).wait()

        for b in range(NBUF):
            gather(b, b)

        lanes = lax.iota(jnp.int32, L)
        perms = [(lanes + k) & (L - 1) for k in range(L)]

        def outer(jj, _):
            for b in range(NBUF):
                j = jj * NBUF + b
                bo = b % NOBUF
                pltpu.make_async_copy(table_hbm.at[idx2[b]], bin_[b],
                                      gsem[b]).wait()

                if b >= NOBUF:
                    put_wait(bo)
                else:
                    @pl.when(jj > 0)
                    def _():
                        put_wait(bo)

                def tblock(blk, _):
                    sv = blk // (D // L)
                    d0 = (blk % (D // L)) * L
                    rows = sv * L + lanes
                    pv = lax.shift_left(
                        jnp.bitwise_and(tok_v[j, pl.ds(sv * L, L)], 1), 6)
                    bd0 = jnp.full((L,), d0, jnp.int32)
                    for k in range(L):
                        drow = bd0 + perms[k]
                        vals = plsc.load_gather(bin_[b], [rows, drow + pv])
                        plsc.store_scatter(bout[bo], [drow, rows], vals * SCALE)
                    return ()

                lax.fori_loop(0, (CHUNK // L) * (D // L), tblock, ())

                put(j, bo)

                @pl.when(jj < nouter - 1)
                def _():
                    gather(j + NBUF, b)
            return ()

        lax.fori_loop(0, nouter, outer, ())
        for b in range(NOBUF):
            put_wait(b)

    return emb


NVB = VOCAB // CHUNK


def _repack():
    mesh = plsc.VectorSubcoreMesh(core_axis_name="c", subcore_axis_name="s")
    RB = 4
    nio = (NVB // NW + RB) // RB

    @functools.partial(
        pl.kernel,
        mesh=mesh,
        compiler_params=pltpu.CompilerParams(
            use_tc_tiling_on_sc=True, needs_layout_passes=False),
        out_type=jax.ShapeDtypeStruct((VOCAB // 2, 2 * D), jnp.float32),
        scratch_types=[
            [pltpu.VMEM((D, CHUNK), jnp.float32) for _ in range(RB)],
            [pltpu.VMEM((CHUNK // 2, 2 * D), jnp.float32) for _ in range(RB)],
            pltpu.VMEM((D, D), jnp.float32),
            pltpu.VMEM((D // 2, 2 * D), jnp.float32),
            [pltpu.SemaphoreType.DMA for _ in range(RB)],
            [pltpu.SemaphoreType.DMA for _ in range(RB)],
        ],
    )
    def rpk(tt_hbm, out_hbm, ibuf, obuf, tin, tout, rsem, wsem):
        wid = lax.axis_index("s") * NC + lax.axis_index("c")

        lanes = lax.iota(jnp.int32, L)
        perms = [(lanes + k) & (L - 1) for k in range(L)]
        floor2 = lax.shift_right_logical(lanes, 1)
        parl64 = lax.shift_left(jnp.bitwise_and(lanes, 1), 6)

        def blk_of(i):
            return wid + NW * i

        def read(i, b):
            pltpu.async_copy(
                tt_hbm.at[:, pl.ds(blk_of(i) * CHUNK, CHUNK)], ibuf[b],
                rsem[b])

        def write(i, b):
            pltpu.async_copy(
                obuf[b], out_hbm.at[pl.ds(blk_of(i) * (CHUNK // 2), CHUNK // 2)],
                wsem[b])

        def wwait(b):
            pltpu.make_async_copy(
                obuf[b], out_hbm.at[pl.ds(0, CHUNK // 2)], wsem[b]).wait()

        def transpose(src, dst, nsub):
            def tblock(blk, _):
                vi0 = (blk // (D // L)) * L
                d0 = (blk % (D // L)) * L
                cols_r = vi0 + lanes
                rows_w = (vi0 // 2) + floor2
                bd0 = jnp.full((L,), d0, jnp.int32)
                for k in range(L):
                    rows_r = bd0 + perms[k]
                    vals = plsc.load_gather(src, [rows_r, cols_r])
                    plsc.store_scatter(dst, [rows_w, parl64 + rows_r], vals)
                return ()

            lax.fori_loop(0, nsub, tblock, ())

        for b in range(RB):
            read(b, b)

        def outer(io, _):
            for b in range(RB):
                i = io * RB + b
                blk = blk_of(i)
                cond = blk < NVB

                @pl.when(cond)
                def _():
                    pltpu.make_async_copy(
                        tt_hbm.at[:, pl.ds(0, CHUNK)], ibuf[b], rsem[b]).wait()

                    @pl.when(io > 0)
                    def _():
                        wwait(b)

                    transpose(ibuf[b], obuf[b], (CHUNK // L) * (D // L))
                    write(i, b)

                @pl.when(blk_of(i + RB) < NVB)
                def _():
                    read(i + RB, b)
            return ()

        lax.fori_loop(0, nio, outer, ())
        for b in range(RB):
            wwait(b)

        @pl.when(wid == 0)
        def _():
            pltpu.sync_copy(tt_hbm.at[:, pl.ds(NVB * CHUNK, D)], tin)
            transpose(tin, tout, (D // L) * (D // L))
            pltpu.sync_copy(
                tout, out_hbm.at[pl.ds(NVB * (CHUNK // 2), D // 2)])

    return rpk


def kernel(tokens, table):
    S, T = tokens.shape
    B = S * T
    tok2d = tokens.T.astype(jnp.int32).reshape(B // CHUNK, CHUNK)
    tbl2 = _repack()(table.T)
    out = _build(S, T)(tok2d, tbl2)
    return out.reshape(T, D, S).transpose(2, 0, 1)


# speedup vs baseline: 1.0727x; 1.0727x over previous
import functools
import math

import jax
import jax.numpy as jnp
from jax import lax
from jax.experimental import pallas as pl
from jax.experimental.pallas import tpu as pltpu
from jax.experimental.pallas import tpu_sc as plsc

VOCAB = 1_000_000
D = 64
SCALE = math.sqrt(D)

_info = plsc.get_sparse_core_info()
NC = _info.num_cores
NS = _info.num_subcores
NW = NC * NS
L = _info.num_lanes

CHUNK = 128
NBUF = 4
NOBUF = 2


def _build(S, T):
    B = S * T
    nrows = B // CHUNK
    spt = S // CHUNK
    per_w = nrows // NW
    assert per_w % NBUF == 0
    nouter = per_w // NBUF

    mesh = plsc.VectorSubcoreMesh(core_axis_name="c", subcore_axis_name="s")

    @functools.partial(
        pl.kernel,
        mesh=mesh,
        compiler_params=pltpu.CompilerParams(
            use_tc_tiling_on_sc=True, needs_layout_passes=False),
        out_type=jax.ShapeDtypeStruct((T * D, S), jnp.float32),
        scratch_types=[
            pltpu.VMEM((per_w, CHUNK), jnp.int32),
            [pltpu.VMEM((CHUNK,), jnp.int32) for _ in range(NBUF)],
            [pltpu.VMEM((CHUNK, 2 * D), jnp.float32) for _ in range(NBUF)],
            [pltpu.VMEM((D, CHUNK), jnp.float32) for _ in range(NOBUF)],
            [pltpu.SemaphoreType.DMA for _ in range(NBUF)],
            [pltpu.SemaphoreType.DMA for _ in range(NOBUF)],
        ],
    )
    def emb(tok_hbm, table_hbm, out_hbm, tok_v, idx2, bin_, bout, gsem, osem):
        wid = lax.axis_index("s") * NC + lax.axis_index("c")
        pltpu.sync_copy(tok_hbm.at[pl.ds(wid * per_w, per_w)], tok_v)
        g0 = wid * per_w

        def gather(j, b):
            for sv in range(CHUNK // L):
                sl = pl.ds(sv * L, L)
                idx2[b][sl] = lax.shift_right_logical(tok_v[j, sl], 1)
            pltpu.async_copy(table_hbm.at[idx2[b]], bin_[b], gsem[b])

        def put(j, b):
            g = g0 + j
            t = g // spt
            s0 = (g % spt) * CHUNK
            pltpu.async_copy(
                bout[b], out_hbm.at[pl.ds(t * D, D), pl.ds(s0, CHUNK)], osem[b])

        def put_wait(b):
            pltpu.make_async_copy(
                bout[b], out_hbm.at[pl.ds(0, D), pl.ds(0, CHUNK)],
                osem[b]).wait()

        for b in range(NBUF):
            gather(b, b)

        lanes = lax.iota(jnp.int32, L)
        perms = [(lanes + k) & (L - 1) for k in range(L)]

        def outer(jj, _):
            for b in range(NBUF):
                j = jj * NBUF + b
                bo = b % NOBUF
                pltpu.make_async_copy(table_hbm.at[idx2[b]], bin_[b],
                                      gsem[b]).wait()

                if b >= NOBUF:
                    put_wait(bo)
                else:
                    @pl.when(jj > 0)
                    def _():
                        put_wait(bo)

                def tblock(blk, _):
                    sv = blk // (D // L)
                    d0 = (blk % (D // L)) * L
                    rows = sv * L + lanes
                    pv = lax.shift_left(
                        jnp.bitwise_and(tok_v[j, pl.ds(sv * L, L)], 1), 6)
                    bd0 = jnp.full((L,), d0, jnp.int32)
                    for k in range(L):
                        drow = bd0 + perms[k]
                        vals = plsc.load_gather(bin_[b], [rows, drow + pv])
                        plsc.store_scatter(bout[bo], [drow, rows], vals * SCALE)
                    return ()

                lax.fori_loop(0, (CHUNK // L) * (D // L), tblock, (),
                              unroll=2)

                put(j, bo)

                @pl.when(jj < nouter - 1)
                def _():
                    gather(j + NBUF, b)
            return ()

        lax.fori_loop(0, nouter, outer, ())
        for b in range(NOBUF):
            put_wait(b)

    return emb


NVB = VOCAB // CHUNK


def _repack():
    mesh = plsc.VectorSubcoreMesh(core_axis_name="c", subcore_axis_name="s")
    RB = 4
    nio = (NVB // NW + RB) // RB

    @functools.partial(
        pl.kernel,
        mesh=mesh,
        compiler_params=pltpu.CompilerParams(
            use_tc_tiling_on_sc=True, needs_layout_passes=False),
        out_type=jax.ShapeDtypeStruct((VOCAB // 2, 2 * D), jnp.float32),
        scratch_types=[
            [pltpu.VMEM((D, CHUNK), jnp.float32) for _ in range(RB)],
            [pltpu.VMEM((CHUNK // 2, 2 * D), jnp.float32) for _ in range(RB)],
            pltpu.VMEM((D, D), jnp.float32),
            pltpu.VMEM((D // 2, 2 * D), jnp.float32),
            [pltpu.SemaphoreType.DMA for _ in range(RB)],
            [pltpu.SemaphoreType.DMA for _ in range(RB)],
        ],
    )
    def rpk(tt_hbm, out_hbm, ibuf, obuf, tin, tout, rsem, wsem):
        wid = lax.axis_index("s") * NC + lax.axis_index("c")

        lanes = lax.iota(jnp.int32, L)
        perms = [(lanes + k) & (L - 1) for k in range(L)]
        floor2 = lax.shift_right_logical(lanes, 1)
        parl64 = lax.shift_left(jnp.bitwise_and(lanes, 1), 6)

        def blk_of(i):
            return wid + NW * i

        def read(i, b):
            pltpu.async_copy(
                tt_hbm.at[:, pl.ds(blk_of(i) * CHUNK, CHUNK)], ibuf[b],
                rsem[b])

        def write(i, b):
            pltpu.async_copy(
                obuf[b], out_hbm.at[pl.ds(blk_of(i) * (CHUNK // 2), CHUNK // 2)],
                wsem[b])

        def wwait(b):
            pltpu.make_async_copy(
                obuf[b], out_hbm.at[pl.ds(0, CHUNK // 2)], wsem[b]).wait()

        def transpose(src, dst, nsub):
            def tblock(blk, _):
                vi0 = (blk // (D // L)) * L
                d0 = (blk % (D // L)) * L
                cols_r = vi0 + lanes
                rows_w = (vi0 // 2) + floor2
                bd0 = jnp.full((L,), d0, jnp.int32)
                for k in range(L):
                    rows_r = bd0 + perms[k]
                    vals = plsc.load_gather(src, [rows_r, cols_r])
                    plsc.store_scatter(dst, [rows_w, parl64 + rows_r], vals)
                return ()

            lax.fori_loop(0, nsub, tblock, (), unroll=2)

        for b in range(RB):
            read(b, b)

        def outer(io, _):
            for b in range(RB):
                i = io * RB + b
                blk = blk_of(i)
                cond = blk < NVB

                @pl.when(cond)
                def _():
                    pltpu.make_async_copy(
                        tt_hbm.at[:, pl.ds(0, CHUNK)], ibuf[b], rsem[b]).wait()

                    @pl.when(io > 0)
                    def _():
                        wwait(b)

                    transpose(ibuf[b], obuf[b], (CHUNK // L) * (D // L))
                    write(i, b)

                @pl.when(blk_of(i + RB) < NVB)
                def _():
                    read(i + RB, b)
            return ()

        lax.fori_loop(0, nio, outer, ())
        for b in range(RB):
            wwait(b)

        @pl.when(wid == 0)
        def _():
            pltpu.sync_copy(tt_hbm.at[:, pl.ds(NVB * CHUNK, D)], tin)
            transpose(tin, tout, (D // L) * (D // L))
            pltpu.sync_copy(
                tout, out_hbm.at[pl.ds(NVB * (CHUNK // 2), D // 2)])

    return rpk


def kernel(tokens, table):
    S, T = tokens.shape
    B = S * T
    tok2d = tokens.T.astype(jnp.int32).reshape(B // CHUNK, CHUNK)
    tbl2 = _repack()(table.T)
    out = _build(S, T)(tok2d, tbl2)
    return out.reshape(T, D, S).transpose(2, 0, 1)
